# baseline (speedup 1.0000x reference)
import functools

import jax
import jax.numpy as jnp
from jax import lax
from jax.experimental import pallas as pl
from jax.experimental.pallas import tpu as pltpu
from jax.experimental.pallas import tpu_sc as plsc

_D = 128
_B = 16384 * 20
_NC = 2
_NS = 16
_NW = _NC * _NS
_BPW = _B // _NW
_CHUNK = 64
_NCHUNK = _BPW // _CHUNK
_NSLOT = 10
_NGROUP = _NCHUNK // _NSLOT


def _sc_gather(x3, W):
    mesh = plsc.VectorSubcoreMesh(core_axis_name="c", subcore_axis_name="s")

    @functools.partial(
        pl.kernel,
        out_type=jax.ShapeDtypeStruct((_B, _D), jnp.float32),
        mesh=mesh,
        scratch_types=[
            pltpu.VMEM((_NCHUNK, _CHUNK), jnp.int32),
            *[pltpu.VMEM((_CHUNK, _D), jnp.float32) for _ in range(_NSLOT)],
            *[pltpu.SemaphoreType.DMA for _ in range(_NSLOT)],
            *[pltpu.SemaphoreType.DMA for _ in range(_NSLOT)],
        ],
    )
    def body(x_hbm, w_hbm, out_hbm, idx_v, *rest):
        bufs = rest[:_NSLOT]
        g_sems = rest[_NSLOT:2 * _NSLOT]
        w_sems = rest[2 * _NSLOT:]
        wid = lax.axis_index("s") * _NC + lax.axis_index("c")
        base = wid * _BPW

        pltpu.sync_copy(x_hbm.at[wid], idx_v)

        def start_gather(g, slot):
            pltpu.async_copy(w_hbm.at[idx_v.at[g]], bufs[slot], g_sems[slot])

        def wait_gather(g, slot):
            pltpu.make_async_copy(
                w_hbm.at[idx_v.at[g]], bufs[slot], g_sems[slot]).wait()

        def start_write(g, slot):
            pltpu.async_copy(
                bufs[slot], out_hbm.at[pl.ds(base + g * _CHUNK, _CHUNK)],
                w_sems[slot])

        def wait_write(g, slot):
            pltpu.make_async_copy(
                bufs[slot], out_hbm.at[pl.ds(base + g * _CHUNK, _CHUNK)],
                w_sems[slot]).wait()

        for b in range(_NSLOT):
            start_gather(b, b)

        for b in range(_NSLOT):
            wait_gather(b, b)
            start_write(b, b)
            start_gather(b + _NSLOT, b)

        def group(i, _):
            for b in range(_NSLOT):
                g = _NSLOT * i + b
                wait_gather(g, b)
                start_write(g, b)
                wait_write(g - _NSLOT, b)
                start_gather(g + _NSLOT, b)
            return 0

        lax.fori_loop(1, _NGROUP - 1, group, 0, unroll=False)

        for b in range(_NSLOT):
            g = _NSLOT * (_NGROUP - 1) + b
            wait_gather(g, b)
            wait_write(g - _NSLOT, b)
            start_write(g, b)
        for b in range(_NSLOT):
            g = _NSLOT * (_NGROUP - 1) + b
            wait_write(g, b)

    return body(x3, W)


def kernel(x, W):
    x3 = x.reshape(_NW, _NCHUNK, _CHUNK).astype(jnp.int32)
    out = _sc_gather(x3, W)
    return out.reshape(x.shape[0], x.shape[1], _D)
